# baseline (speedup 1.0000x reference)
import functools

import jax
import jax.numpy as jnp
from jax import lax
from jax.experimental import pallas as pl
from jax.experimental.pallas import tpu as pltpu
from jax.experimental.pallas import tpu_sc as plsc

_NC = 2
_NS = 16
_LANES = 16
_NW = _NC * _NS


def kernel(token_ids, idf_weights):
    B, L = token_ids.shape
    V = idf_weights.shape[0]

    rows_per_w = B // _NW
    groups = rows_per_w // _LANES
    n_full = V // _LANES
    tail_off = V - _LANES

    mesh = plsc.VectorSubcoreMesh(core_axis_name="c", subcore_axis_name="s")

    @functools.partial(
        pl.kernel,
        out_type=jax.ShapeDtypeStruct((B, V), jnp.float32),
        mesh=mesh,
        compiler_params=pltpu.CompilerParams(needs_layout_passes=False),
        scratch_types=[
            pltpu.VMEM((_LANES, L), jnp.int32),
            pltpu.VMEM((V,), jnp.float32),
            pltpu.VMEM((_LANES, V), jnp.float32),
        ],
    )
    def _tfidf(tok_hbm, idf_hbm, out_hbm, tok_v, idf_v, acc_v):
        wid = lax.axis_index("s") * _NC + lax.axis_index("c")
        base = wid * rows_per_w
        pltpu.sync_copy(idf_hbm, idf_v)
        lanes = lax.iota(jnp.int32, _LANES)
        zeros = jnp.zeros((_LANES,), jnp.float32)

        def group_body(g, carry):
            row0 = base + g * _LANES
            pltpu.sync_copy(tok_hbm.at[pl.ds(row0, _LANES), :], tok_v)

            @plsc.parallel_loop(0, n_full, 1, unroll=4)
            def _zero(c):
                off = pl.multiple_of(c * _LANES, _LANES)
                for l in range(_LANES):
                    acc_v[l, pl.ds(off, _LANES)] = zeros

            for l in range(_LANES):
                acc_v[l, pl.ds(tail_off, _LANES)] = zeros

            @plsc.parallel_loop(0, L, 1, unroll=8)
            def _tok(j):
                jv = jnp.full((_LANES,), j, jnp.int32)
                tok = plsc.load_gather(tok_v, [lanes, jv])
                val = plsc.load_gather(idf_v, [tok])
                plsc.addupdate_scatter(acc_v, [lanes, tok], val)

            pltpu.sync_copy(acc_v, out_hbm.at[pl.ds(row0, _LANES), :])
            return carry

        lax.fori_loop(0, groups, group_body, 0, unroll=False)

    return _tfidf(token_ids, idf_weights)


# speedup vs baseline: 1.1479x; 1.1479x over previous
import functools

import jax
import jax.numpy as jnp
from jax import lax
from jax.experimental import pallas as pl
from jax.experimental.pallas import tpu as pltpu
from jax.experimental.pallas import tpu_sc as plsc

_NC = 2
_NS = 16
_LANES = 16
_NW = _NC * _NS


def kernel(token_ids, idf_weights):
    B, L = token_ids.shape
    V = idf_weights.shape[0]

    rows_per_w = B // _NW
    groups = rows_per_w // _LANES
    n_full = V // _LANES
    tail_off = V - _LANES

    mesh = plsc.VectorSubcoreMesh(core_axis_name="c", subcore_axis_name="s")

    @functools.partial(
        pl.kernel,
        out_type=jax.ShapeDtypeStruct((B, V), jnp.float32),
        mesh=mesh,
        compiler_params=pltpu.CompilerParams(needs_layout_passes=False),
        scratch_types=[
            pltpu.VMEM((_LANES, L), jnp.int32),
            pltpu.VMEM((_LANES, L), jnp.int32),
            pltpu.VMEM((V,), jnp.float32),
            pltpu.VMEM((_LANES, V), jnp.float32),
            pltpu.VMEM((_LANES, V), jnp.float32),
            pltpu.SemaphoreType.DMA,
            pltpu.SemaphoreType.DMA,
            pltpu.SemaphoreType.DMA,
            pltpu.SemaphoreType.DMA,
        ],
    )
    def _tfidf(tok_hbm, idf_hbm, out_hbm, tok0_v, tok1_v, idf_v,
               acc0_v, acc1_v, st0, st1, so0, so1):
        tok_v = (tok0_v, tok1_v)
        acc_v = (acc0_v, acc1_v)
        st = (st0, st1)
        so = (so0, so1)

        wid = lax.axis_index("s") * _NC + lax.axis_index("c")
        base = wid * rows_per_w
        lanes = lax.iota(jnp.int32, _LANES)
        zeros = jnp.zeros((_LANES,), jnp.float32)

        tok_dma = [None, None]
        out_dma = [None, None]

        tok_dma[0] = pltpu.async_copy(
            tok_hbm.at[pl.ds(base, _LANES), :], tok_v[0], st[0])
        pltpu.sync_copy(idf_hbm, idf_v)

        def zero_acc(acc):
            @plsc.parallel_loop(0, n_full, 1, unroll=4)
            def _zero(c):
                off = pl.multiple_of(c * _LANES, _LANES)
                for l in range(_LANES):
                    acc[l, pl.ds(off, _LANES)] = zeros

            for l in range(_LANES):
                acc[l, pl.ds(tail_off, _LANES)] = zeros

        zero_acc(acc_v[0])
        zero_acc(acc_v[1])

        for g in range(groups):
            b = g & 1
            row0 = base + g * _LANES
            if g + 1 < groups:
                tok_dma[1 - b] = pltpu.async_copy(
                    tok_hbm.at[pl.ds(row0 + _LANES, _LANES), :],
                    tok_v[1 - b], st[1 - b])
            tok_dma[b].wait()

            @plsc.parallel_loop(0, L, 1, unroll=8)
            def _tok(j):
                jv = jnp.full((_LANES,), j, jnp.int32)
                tok = plsc.load_gather(tok_v[b], [lanes, jv])
                val = plsc.load_gather(idf_v, [tok])
                plsc.addupdate_scatter(acc_v[b], [lanes, tok], val)

            out_dma[b] = pltpu.async_copy(
                acc_v[b], out_hbm.at[pl.ds(row0, _LANES), :], so[b])
            if g >= 1 and g + 1 < groups:
                out_dma[1 - b].wait()
                zero_acc(acc_v[1 - b])

        out_dma[(groups - 2) & 1].wait()
        out_dma[(groups - 1) & 1].wait()

    return _tfidf(token_ids, idf_weights)


# speedup vs baseline: 1.4055x; 1.2244x over previous
import functools

import jax
import jax.numpy as jnp
from jax import lax
from jax.experimental import pallas as pl
from jax.experimental.pallas import tpu as pltpu
from jax.experimental.pallas import tpu_sc as plsc

_NC = 2
_NS = 16
_LANES = 16
_NW = _NC * _NS
_WCOLS = 128
_CHUNK = 8


def kernel(token_ids, idf_weights):
    B, L = token_ids.shape
    V = idf_weights.shape[0]

    n_chunks = L // _CHUNK
    col_groups = _WCOLS // _LANES

    mesh = plsc.VectorSubcoreMesh(core_axis_name="c", subcore_axis_name="s")

    @functools.partial(
        pl.kernel,
        out_type=jax.ShapeDtypeStruct((V, B), jnp.float32),
        mesh=mesh,
        compiler_params=pltpu.CompilerParams(needs_layout_passes=False),
        scratch_types=[
            pltpu.VMEM((_CHUNK, _WCOLS), jnp.int32),
            pltpu.VMEM((_CHUNK, _WCOLS), jnp.int32),
            pltpu.VMEM((V,), jnp.float32),
            pltpu.VMEM((V, _WCOLS), jnp.float32),
            pltpu.SemaphoreType.DMA,
            pltpu.SemaphoreType.DMA,
        ],
    )
    def _tfidf(tok_hbm, idf_hbm, out_hbm, tokc0, tokc1, idf_v, acc_v,
               sc0, sc1):
        cid = lax.axis_index("c")
        sid = lax.axis_index("s")
        wid = sid * _NC + cid
        wbase = pl.multiple_of(wid * _WCOLS, _WCOLS)
        lanes = lax.iota(jnp.int32, _LANES)
        zeros = jnp.zeros((_LANES,), jnp.float32)
        tokc = (tokc0, tokc1)
        scs = (sc0, sc1)

        chunk_dma = [None, None]
        chunk_dma[0] = pltpu.async_copy(
            tok_hbm.at[pl.ds(0, _CHUNK), pl.ds(wbase, _WCOLS)], tokc[0], scs[0])
        pltpu.sync_copy(idf_hbm, idf_v)

        @plsc.parallel_loop(0, V, 1, unroll=2)
        def _zero(v):
            for c in range(col_groups):
                acc_v[v, pl.ds(c * _LANES, _LANES)] = zeros

        for k in range(n_chunks):
            b = k & 1
            if k + 1 < n_chunks:
                chunk_dma[1 - b] = pltpu.async_copy(
                    tok_hbm.at[pl.ds((k + 1) * _CHUNK, _CHUNK),
                               pl.ds(wbase, _WCOLS)],
                    tokc[1 - b], scs[1 - b])
            chunk_dma[b].wait()

            @plsc.parallel_loop(0, _CHUNK, 1, unroll=2)
            def _row(j):
                for c in range(col_groups):
                    colv = lanes + (c * _LANES)
                    tok = tokc[b][j, pl.ds(c * _LANES, _LANES)]
                    val = plsc.load_gather(idf_v, [tok])
                    plsc.addupdate_scatter(acc_v, [tok, colv], val)

        pltpu.sync_copy(acc_v, out_hbm.at[:, pl.ds(wbase, _WCOLS)])

    return _tfidf(token_ids.T, idf_weights).T


# speedup vs baseline: 1.7264x; 1.2283x over previous
import functools

import jax
import jax.numpy as jnp
from jax import lax
from jax.experimental import pallas as pl
from jax.experimental.pallas import tpu as pltpu
from jax.experimental.pallas import tpu_sc as plsc

_NC = 2
_NS = 16
_LANES = 16
_NW = _NC * _NS
_WCOLS = 128
_CHUNK = 8


def kernel(token_ids, idf_weights):
    B, L = token_ids.shape
    V = idf_weights.shape[0]

    n_chunks = L // _CHUNK
    col_groups = _WCOLS // _LANES

    mesh = plsc.VectorSubcoreMesh(core_axis_name="c", subcore_axis_name="s")

    @functools.partial(
        pl.kernel,
        out_type=jax.ShapeDtypeStruct((V, B), jnp.float32),
        mesh=mesh,
        compiler_params=pltpu.CompilerParams(needs_layout_passes=False),
        scratch_types=[
            pltpu.VMEM((_CHUNK, _WCOLS), jnp.int32),
            pltpu.VMEM((_CHUNK, _WCOLS), jnp.int32),
            pltpu.VMEM((V,), jnp.float32),
            pltpu.VMEM((V, _WCOLS), jnp.float32),
            pltpu.SemaphoreType.DMA,
            pltpu.SemaphoreType.DMA,
        ],
    )
    def _tfidf(tok_hbm, idf_hbm, out_hbm, tokc0, tokc1, idf_v, acc_v,
               sc0, sc1):
        cid = lax.axis_index("c")
        sid = lax.axis_index("s")
        wid = sid * _NC + cid
        wbase = pl.multiple_of(wid * _WCOLS, _WCOLS)
        lanes = lax.iota(jnp.int32, _LANES)
        zeros = jnp.zeros((_LANES,), jnp.float32)
        tokc = (tokc0, tokc1)
        scs = (sc0, sc1)

        chunk_dma = [None, None]
        chunk_dma[0] = pltpu.async_copy(
            tok_hbm.at[pl.ds(0, _CHUNK), pl.ds(wbase, _WCOLS)], tokc[0], scs[0])
        pltpu.sync_copy(idf_hbm, idf_v)

        @plsc.parallel_loop(0, V, 1, unroll=2)
        def _zero(v):
            for c in range(col_groups):
                acc_v[v, pl.ds(c * _LANES, _LANES)] = zeros

        chunk_dma[1] = pltpu.async_copy(
            tok_hbm.at[pl.ds(_CHUNK, _CHUNK), pl.ds(wbase, _WCOLS)],
            tokc[1], scs[1])

        def _process(buf):
            @plsc.parallel_loop(0, _CHUNK, 1, unroll=2)
            def _row(j):
                for c in range(col_groups):
                    colv = lanes + (c * _LANES)
                    tok = buf[j, pl.ds(c * _LANES, _LANES)]
                    val = plsc.load_gather(idf_v, [tok])
                    plsc.addupdate_scatter(acc_v, [tok, colv], val)

        def _pair(g, carry):
            for b in range(2):
                k = g * 2 + b
                pltpu.make_async_copy(
                    tok_hbm.at[pl.ds(pl.multiple_of(k * _CHUNK, _CHUNK),
                                     _CHUNK), pl.ds(wbase, _WCOLS)],
                    tokc[b], scs[b]).wait()
                _process(tokc[b])

                @pl.when(k + 2 < n_chunks)
                def _prefetch():
                    pltpu.async_copy(
                        tok_hbm.at[pl.ds(pl.multiple_of((k + 2) * _CHUNK,
                                                        _CHUNK), _CHUNK),
                                   pl.ds(wbase, _WCOLS)],
                        tokc[b], scs[b])
            return carry

        lax.fori_loop(0, (n_chunks - 1) // 2, _pair, 0, unroll=False)

        last = n_chunks - 1
        pltpu.make_async_copy(
            tok_hbm.at[pl.ds(last * _CHUNK, _CHUNK), pl.ds(wbase, _WCOLS)],
            tokc[0], scs[0]).wait()
        _process(tokc[0])

        pltpu.sync_copy(acc_v, out_hbm.at[:, pl.ds(wbase, _WCOLS)])

    return _tfidf(token_ids.T, idf_weights).T
